# baseline (speedup 1.0000x reference)
import functools
import math

import jax
import jax.numpy as jnp
from jax import lax
from jax.experimental import pallas as pl
from jax.experimental.pallas import tpu as pltpu
from jax.experimental.pallas import tpu_sc as plsc

_C2 = 2.0 * math.sqrt(2.0 / math.pi)
_A2 = _C2 * 0.044715
_N = 2 * 4096 * 4096
_NC = 2
_NS = 16
_NW = _NC * _NS
_PER_W = _N // _NW
_CH = 16384
_NCHUNK = _PER_W // _CH

_mesh = plsc.VectorSubcoreMesh(core_axis_name="c", subcore_axis_name="s")


@functools.partial(
    pl.kernel,
    mesh=_mesh,
    out_type=jax.ShapeDtypeStruct((_N,), jnp.float32),
    scratch_types=[pltpu.VMEM((_CH,), jnp.float32)],
)
def _sc_gelu(x_hbm, o_hbm, buf):
    wid = lax.axis_index("s") * _NC + lax.axis_index("c")
    base = wid * _PER_W

    def chunk_body(ci, _):
        off = base + ci * _CH
        pltpu.sync_copy(x_hbm.at[pl.ds(off, _CH)], buf)

        def inner(j, _):
            v = buf[pl.ds(j * 16, 16)]
            u = v * v
            z2 = v * (_C2 + _A2 * u)
            e = jnp.exp(-z2)
            buf[pl.ds(j * 16, 16)] = v / (1.0 + e)
            return 0

        lax.fori_loop(0, _CH // 16, inner, 0)
        pltpu.sync_copy(buf, o_hbm.at[pl.ds(off, _CH)])
        return 0

    lax.fori_loop(0, _NCHUNK, chunk_body, 0)


def kernel(x, log_k_blend):
    del log_k_blend
    out = _sc_gelu(x.reshape(_N))
    return out.reshape(x.shape)


# speedup vs baseline: 23.9717x; 23.9717x over previous
import math

import jax
import jax.numpy as jnp
from jax import lax
from jax.experimental import pallas as pl
from jax.experimental.pallas import tpu as pltpu

_C = math.sqrt(2.0 / math.pi)
_A = _C * 0.044715
_ROWS = 8192
_COLS = 4096
_CH_ROWS = 64
_NCH = _ROWS // _CH_ROWS
_NBUF = 12


def _gelu(x):
    u = x * x
    z = x * (_C + _A * u)
    h = 0.5 * x
    return h + h * jnp.tanh(z)


def _pipelined_gelu(x_hbm, o_hbm, ibuf, obuf, in_sems, out_sems):
    def copy_in(i, slot):
        return pltpu.make_async_copy(
            x_hbm.at[pl.ds(i * _CH_ROWS, _CH_ROWS), :],
            ibuf.at[slot],
            in_sems.at[slot],
        )

    def copy_out(i, slot):
        return pltpu.make_async_copy(
            obuf.at[slot],
            o_hbm.at[pl.ds(i * _CH_ROWS, _CH_ROWS), :],
            out_sems.at[slot],
        )

    for s in range(_NBUF):
        copy_in(s, s).start()

    def body(i, _):
        slot = lax.rem(i, _NBUF)
        copy_in(i, slot).wait()

        @pl.when(i >= _NBUF)
        def _():
            copy_out(i - _NBUF, slot).wait()

        obuf[slot] = ibuf[slot]
        copy_out(i, slot).start()

        @pl.when(i + _NBUF < _NCH)
        def _():
            copy_in(i + _NBUF, slot).start()

        return 0

    lax.fori_loop(0, _NCH, body, 0)

    for s in range(_NBUF):
        i = _NCH - _NBUF + s
        copy_out(i, lax.rem(jnp.int32(i), _NBUF)).wait()


def kernel(x, log_k_blend):
    del log_k_blend
    x2 = x.reshape(_ROWS, _COLS)
    out = pl.pallas_call(
        _pipelined_gelu,
        in_specs=[pl.BlockSpec(memory_space=pl.ANY)],
        out_specs=pl.BlockSpec(memory_space=pl.ANY),
        out_shape=jax.ShapeDtypeStruct((_ROWS, _COLS), jnp.float32),
        scratch_shapes=[
            pltpu.VMEM((_NBUF, _CH_ROWS, _COLS), jnp.float32),
            pltpu.VMEM((_NBUF, _CH_ROWS, _COLS), jnp.float32),
            pltpu.SemaphoreType.DMA((_NBUF,)),
            pltpu.SemaphoreType.DMA((_NBUF,)),
        ],
        compiler_params=pltpu.CompilerParams(
            vmem_limit_bytes=120 * 1024 * 1024,
        ),
    )(x2)
    return out.reshape(x.shape)


# speedup vs baseline: 24.0096x; 1.0016x over previous
import math

import jax
import jax.numpy as jnp
from jax import lax
from jax.experimental import pallas as pl
from jax.experimental.pallas import tpu as pltpu

_C = math.sqrt(2.0 / math.pi)
_A = _C * 0.044715
_ROWS = 8192
_COLS = 4096
_CH_ROWS = 64
_NCH = _ROWS // _CH_ROWS
_NBUF = 12


def _gelu(x):
    u = x * x
    z = x * (_C + _A * u)
    h = 0.5 * x
    return h + h * jnp.tanh(z)


def _pipelined_gelu(x_hbm, o_hbm, ibuf, obuf, in_sems, out_sems):
    def copy_in(i, slot):
        return pltpu.make_async_copy(
            x_hbm.at[pl.ds(i * _CH_ROWS, _CH_ROWS), :],
            ibuf.at[slot],
            in_sems.at[slot],
        )

    def copy_out(i, slot):
        return pltpu.make_async_copy(
            obuf.at[slot],
            o_hbm.at[pl.ds(i * _CH_ROWS, _CH_ROWS), :],
            out_sems.at[slot],
        )

    for s in range(_NBUF):
        copy_in(s, s).start()

    def body(i, _):
        slot = lax.rem(i, _NBUF)
        copy_in(i, slot).wait()

        @pl.when(i >= _NBUF)
        def _():
            copy_out(i - _NBUF, slot).wait()

        obuf[slot] = _gelu(ibuf[slot])
        copy_out(i, slot).start()

        @pl.when(i + _NBUF < _NCH)
        def _():
            copy_in(i + _NBUF, slot).start()

        return 0

    lax.fori_loop(0, _NCH, body, 0)

    for s in range(_NBUF):
        i = _NCH - _NBUF + s
        copy_out(i, lax.rem(jnp.int32(i), _NBUF)).wait()


def kernel(x, log_k_blend):
    del log_k_blend
    x2 = x.reshape(_ROWS, _COLS)
    out = pl.pallas_call(
        _pipelined_gelu,
        in_specs=[pl.BlockSpec(memory_space=pl.ANY)],
        out_specs=pl.BlockSpec(memory_space=pl.ANY),
        out_shape=jax.ShapeDtypeStruct((_ROWS, _COLS), jnp.float32),
        scratch_shapes=[
            pltpu.VMEM((_NBUF, _CH_ROWS, _COLS), jnp.float32),
            pltpu.VMEM((_NBUF, _CH_ROWS, _COLS), jnp.float32),
            pltpu.SemaphoreType.DMA((_NBUF,)),
            pltpu.SemaphoreType.DMA((_NBUF,)),
        ],
        compiler_params=pltpu.CompilerParams(
            vmem_limit_bytes=120 * 1024 * 1024,
        ),
    )(x2)
    return out.reshape(x.shape)
